# baseline (speedup 1.0000x reference)
import functools

import jax
import jax.numpy as jnp
from jax import lax
from jax.experimental import pallas as pl
from jax.experimental.pallas import tpu as pltpu
from jax.experimental.pallas import tpu_sc as plsc

V = 10000
E = 160000
K = 5
CL1_F = 32
POOL = 8
B = 64
NPOOL = V // POOL
FC1_IN = NPOOL * CL1_F


_VBLK = 400
_PBLK = _VBLK // POOL


def _cheby_pool_body(xc_ref, w_ref, b_ref, out_ref):
    x = xc_ref[...]
    w = w_ref[...]
    pre = lax.dot_general(x, w, (((0,), (0,)), ((), ())))
    pre = pre.reshape(_PBLK, POOL, B, CL1_F)
    pooled = jnp.max(pre, axis=1)
    act = jnp.maximum(pooled + b_ref[...][None, :, :].reshape(1, 1, CL1_F), 0.0)
    out_ref[...] = jnp.transpose(act, (0, 2, 1))


def _cheby_pool(xc2, W_cl1, b_cl1):
    grid = V // _VBLK
    out = pl.pallas_call(
        _cheby_pool_body,
        grid=(grid,),
        in_specs=[
            pl.BlockSpec((K, _VBLK * B), lambda i: (0, i)),
            pl.BlockSpec((K, CL1_F), lambda i: (0, 0)),
            pl.BlockSpec((1, CL1_F), lambda i: (0, 0)),
        ],
        out_specs=pl.BlockSpec((_PBLK, CL1_F, B), lambda i: (i, 0, 0)),
        out_shape=jax.ShapeDtypeStruct((NPOOL, CL1_F, B), jnp.float32),
    )(xc2, W_cl1, b_cl1.reshape(1, CL1_F))
    return out.reshape(FC1_IN, B)


_IBLK = 2000


def _fc1_body(xp_ref, w_ref, b_ref, out_ref, acc_ref):
    @pl.when(pl.program_id(0) == 0)
    def _():
        acc_ref[...] = jnp.zeros_like(acc_ref)

    acc_ref[...] += lax.dot_general(
        w_ref[...], xp_ref[...], (((0,), (0,)), ((), ()))
    )

    @pl.when(pl.program_id(0) == pl.num_programs(0) - 1)
    def _():
        out_ref[...] = jnp.maximum(acc_ref[...] + b_ref[...], 0.0)


def _fc1(xpT, W_fc1, b_fc1):
    grid = FC1_IN // _IBLK
    return pl.pallas_call(
        _fc1_body,
        grid=(grid,),
        in_specs=[
            pl.BlockSpec((_IBLK, B), lambda i: (i, 0)),
            pl.BlockSpec((_IBLK, 512), lambda i: (i, 0)),
            pl.BlockSpec((512, 1), lambda i: (0, 0)),
        ],
        out_specs=pl.BlockSpec((512, B), lambda i: (0, 0)),
        out_shape=jax.ShapeDtypeStruct((512, B), jnp.float32),
        scratch_shapes=[pltpu.VMEM((512, B), jnp.float32)],
    )(xpT, W_fc1, b_fc1.reshape(512, 1))


def _fc2_body(h_ref, w_ref, b_ref, out_ref):
    out_ref[...] = jnp.maximum(
        lax.dot_general(w_ref[...], h_ref[...], (((0,), (0,)), ((), ())))
        + b_ref[...],
        0.0,
    )


def _fc2(hT, W_fc2, b_fc2):
    return pl.pallas_call(
        _fc2_body,
        out_shape=jax.ShapeDtypeStruct((1024, B), jnp.float32),
    )(hT, W_fc2, b_fc2.reshape(1024, 1))


_KBLK = 128


def _fc3_body(d_ref, w_ref, b_ref, out_ref, acc_ref):
    @pl.when(pl.program_id(0) == 0)
    def _():
        acc_ref[...] = jnp.zeros_like(acc_ref)

    acc_ref[...] += lax.dot_general(
        d_ref[...], w_ref[...], (((0,), (0,)), ((), ()))
    )

    @pl.when(pl.program_id(0) == pl.num_programs(0) - 1)
    def _():
        out_ref[...] = acc_ref[...] + b_ref[...]


def _fc3(d2T, W_fc3, b_fc3):
    grid = 1024 // _KBLK
    return pl.pallas_call(
        _fc3_body,
        grid=(grid,),
        in_specs=[
            pl.BlockSpec((_KBLK, B), lambda i: (i, 0)),
            pl.BlockSpec((_KBLK, V), lambda i: (i, 0)),
            pl.BlockSpec((1, V), lambda i: (0, 0)),
        ],
        out_specs=pl.BlockSpec((B, V), lambda i: (0, 0)),
        out_shape=jax.ShapeDtypeStruct((B, V), jnp.float32),
        scratch_shapes=[pltpu.VMEM((B, V), jnp.float32)],
    )(d2T, W_fc3, b_fc3.reshape(1, V))


def _log_softmax(x):
    m = jnp.max(x, axis=1, keepdims=True)
    s = x - m
    return s - jnp.log(jnp.sum(jnp.exp(s), axis=1, keepdims=True))


def _heads_body(h_ref, xnn_ref, wn1_ref, bn1_ref, wn2_ref, bn2_ref,
                ws2_ref, bs2_ref, wi1_ref, bi1_ref, wi2_ref, bi2_ref,
                h_out, xn_out, xout_out, gae_out, fc_out):
    hT = h_ref[...]
    a1 = jnp.maximum(
        lax.dot_general(wn1_ref[...], xnn_ref[...], (((0,), (0,)), ((), ())))
        + bn1_ref[...],
        0.0,
    )
    xnT = jnp.maximum(
        lax.dot_general(wn2_ref[...], a1, (((0,), (0,)), ((), ())))
        + bn2_ref[...],
        0.0,
    )
    h_out[...] = hT.T
    xn_out[...] = xnT.T
    xcatT = jnp.concatenate([hT, xnT], axis=0)
    lg = lax.dot_general(xcatT, ws2_ref[...], (((0,), (0,)), ((), ()))) + bs2_ref[...]
    xout_out[...] = _log_softmax(lg)
    g1 = lax.dot_general(hT, wi1_ref[...], (((0,), (0,)), ((), ()))) + bi1_ref[...]
    gae_out[...] = _log_softmax(jnp.dot(g1, wi2_ref[...]) + bi2_ref[...])
    f1 = lax.dot_general(xnT, wi1_ref[...], (((0,), (0,)), ((), ()))) + bi1_ref[...]
    fc_out[...] = _log_softmax(jnp.dot(f1, wi2_ref[...]) + bi2_ref[...])


def _heads(hT, x_nnT, W_nn1, b_nn1, W_nn2, b_nn2, W_sum2, b_sum2,
           W_im1, b_im1, W_im2, b_im2):
    return pl.pallas_call(
        _heads_body,
        out_shape=(
            jax.ShapeDtypeStruct((B, 512), jnp.float32),
            jax.ShapeDtypeStruct((B, 512), jnp.float32),
            jax.ShapeDtypeStruct((B, 27), jnp.float32),
            jax.ShapeDtypeStruct((B, 27), jnp.float32),
            jax.ShapeDtypeStruct((B, 27), jnp.float32),
        ),
    )(hT, x_nnT, W_nn1, b_nn1.reshape(1024, 1), W_nn2, b_nn2.reshape(512, 1),
      W_sum2, b_sum2.reshape(1, 27), W_im1, b_im1.reshape(1, 256),
      W_im2, b_im2.reshape(1, 27))


_NTILES = 16
_EPT = E // _NTILES
_VPAD = 10240
_SLICE = _VPAD // _NTILES
_POWER_ITERS = 50


def _inv_norm(n2):
    x = jnp.full((16,), n2, jnp.float32)

    def sb(i, s):
        return 0.5 * (s + x / s)
    s = lax.fori_loop(0, 34, sb, jnp.maximum(x, 1e-12))
    return 1.0 / (s + 1e-12)


def _power_body(er_hbm, ec_hbm, ew_hbm, lmax_hbm, rows_v, cols_v, w_v, xloc,
                yloc, idx_v, sl_v, sl2_v, red_v, part_v, y_sh, part_s, sem):
    cid = lax.axis_index("c")
    sid = lax.axis_index("s")
    base = sid * _EPT
    sbase = sid * _SLICE

    pltpu.sync_copy(er_hbm.at[pl.ds(base, _EPT)], rows_v)
    pltpu.sync_copy(ec_hbm.at[pl.ds(base, _EPT)], cols_v)
    pltpu.sync_copy(ew_hbm.at[pl.ds(base, _EPT)], w_v)

    def _fill(ref, n, val):
        def fb(i, _):
            ref[pl.ds(i * 16, 16)] = jnp.full((16,), val, jnp.float32)
            return 0
        lax.fori_loop(0, n // 16, fb, 0)

    _fill(xloc, _VPAD, 1.0)
    for j in range((_VPAD - V) // 16):
        xloc[pl.ds(V + j * 16, 16)] = jnp.zeros((16,), jnp.float32)
    _fill(sl_v, _SLICE, 0.0)

    def ib(i, _):
        idx_v[pl.ds(i * 16, 16)] = lax.iota(jnp.int32, 16) + i * 16
        return 0
    lax.fori_loop(0, V // 16, ib, 0)

    def _spmm_step(inv):
        def zb(i, _):
            yloc[pl.ds(i * 16, 16)] = jnp.zeros((16,), jnp.float32)
            return 0
        lax.fori_loop(0, V // 16, zb, 0)
        pltpu.sync_copy(sl_v, y_sh.at[pl.ds(sbase, _SLICE)])

        def eb(i, _):
            o = i * 16
            idx = cols_v[pl.ds(o, 16)]
            val = plsc.load_gather(xloc, [idx]) * w_v[pl.ds(o, 16)] * inv
            plsc.addupdate_scatter(yloc, [rows_v[pl.ds(o, 16)]], val)
            return 0
        lax.fori_loop(0, _EPT // 16, eb, 0)
        plsc.subcore_barrier()
        pltpu.sync_copy(yloc, y_sh.at[idx_v], add=True)
        plsc.subcore_barrier()

    def _sum_partials():
        pltpu.sync_copy(part_v, part_s.at[pl.ds(sid * 16, 16)])
        plsc.subcore_barrier()
        pltpu.sync_copy(part_s, red_v)

        def rb(i, acc):
            return acc + red_v[pl.ds(i * 16, 16)]
        tot = lax.fori_loop(0, _NTILES, rb, jnp.zeros((16,), jnp.float32))
        s = tot[0]
        for j in range(1, 16):
            s = s + tot[j]
        return s

    def _step(t, inv):
        _spmm_step(inv)
        pltpu.sync_copy(y_sh.at[pl.ds(sbase, _SLICE)], sl2_v)

        def nb(i, acc):
            x = sl2_v[pl.ds(i * 16, 16)]
            return acc + x * x
        part_v[...] = lax.fori_loop(
            0, _SLICE // 16, nb, jnp.zeros((16,), jnp.float32))
        inv = _inv_norm(_sum_partials())
        pltpu.sync_copy(y_sh.at[pl.ds(0, _VPAD)], xloc)
        plsc.subcore_barrier()
        return inv

    inv = lax.fori_loop(0, _POWER_ITERS, _step,
                        jnp.ones((16,), jnp.float32))

    _spmm_step(inv)
    pltpu.sync_copy(y_sh.at[pl.ds(sbase, _SLICE)], sl2_v)

    def db(i, acc):
        o = i * 16
        return acc + xloc[pl.ds(sbase + o, 16)] * inv * sl2_v[pl.ds(o, 16)]
    part_v[...] = lax.fori_loop(
        0, _SLICE // 16, db, jnp.zeros((16,), jnp.float32))
    lmax = _sum_partials()

    @pl.when(jnp.logical_and(cid == 0, sid == 0))
    def _():
        part_v[...] = jnp.zeros((16,), jnp.float32) + lmax
        pltpu.sync_copy(part_v, lmax_hbm)


def _power_lmax(edge_rows, edge_cols, edge_weight):
    mesh = plsc.VectorSubcoreMesh(core_axis_name="c", subcore_axis_name="s")
    f = pl.kernel(
        _power_body,
        out_type=jax.ShapeDtypeStruct((16,), jnp.float32),
        mesh=mesh,
        scratch_types=[
            pltpu.VMEM((_EPT,), jnp.int32),
            pltpu.VMEM((_EPT,), jnp.int32),
            pltpu.VMEM((_EPT,), jnp.float32),
            pltpu.VMEM((_VPAD,), jnp.float32),
            pltpu.VMEM((V,), jnp.float32),
            pltpu.VMEM((V,), jnp.int32),
            pltpu.VMEM((_SLICE,), jnp.float32),
            pltpu.VMEM((_SLICE,), jnp.float32),
            pltpu.VMEM((_NTILES * 16,), jnp.float32),
            pltpu.VMEM((16,), jnp.float32),
            pltpu.VMEM_SHARED((_VPAD,), jnp.float32),
            pltpu.VMEM_SHARED((_NTILES * 16,), jnp.float32),
            pltpu.SemaphoreType.DMA,
        ],
        compiler_params=pltpu.CompilerParams(
            use_tc_tiling_on_sc=False, needs_layout_passes=False),
    )
    return f(edge_rows, edge_cols, edge_weight)


_VP = 10000
_VSL = _VP // 16
_CEPT = 5120
_CCHUNK = 1024
_CNCH = _CEPT // _CCHUNK
_CROWS = 125
_CNQ = _VSL // _CROWS


def _spmm_sc_body(x_hbm, er_hbm, ec_hbm, ew_hbm, yp0_hbm, yp1_hbm,
                  r0_v, r1_v, r2_v, r3_v, r4_v, cols_v, w_v, rowbuf, zbuf,
                  y_s, sem):
    cid = lax.axis_index("c")
    sid = lax.axis_index("s")
    ebase = (cid * 16 + sid) * _CEPT
    rows_refs = [r0_v, r1_v, r2_v, r3_v, r4_v]

    for c in range(_CNCH):
        pltpu.sync_copy(er_hbm.at[pl.ds(ebase + c * _CCHUNK, _CCHUNK)],
                        rows_refs[c])
    pltpu.sync_copy(ec_hbm.at[pl.ds(ebase, _CEPT)], cols_v)
    pltpu.sync_copy(ew_hbm.at[pl.ds(ebase, _CEPT)], w_v)

    def zb(r, _):
        for j in range(4):
            zbuf[r, pl.ds(j * 16, 16)] = jnp.zeros((16,), jnp.float32)
        return 0
    lax.fori_loop(0, _CROWS, zb, 0)
    for q in range(_CNQ):
        pltpu.sync_copy(zbuf, y_s.at[pl.ds(sid * _VSL + q * _CROWS, _CROWS)])
    plsc.subcore_barrier()

    for c in range(_CNCH):
        pltpu.async_copy(
            x_hbm.at[cols_v.at[pl.ds(c * _CCHUNK, _CCHUNK)]], rowbuf, sem
        ).wait()

        def gb(g, _):
            w16 = w_v[pl.ds(c * _CCHUNK + g * 16, 16)]
            r0 = g * 16
            for l in range(16):
                wl = jnp.zeros((16,), jnp.float32) + w16[l]
                for fb in range(4):
                    sl = pl.ds(fb * 16, 16)
                    rowbuf[r0 + l, sl] = rowbuf[r0 + l, sl] * wl
            return 0
        lax.fori_loop(0, _CCHUNK // 16, gb, 0)
        pltpu.sync_copy(rowbuf, y_s.at[rows_refs[c]], add=True)
    plsc.subcore_barrier()

    @pl.when(cid == 0)
    def _():
        for q in range(_CNQ):
            o = sid * _VSL + q * _CROWS
            pltpu.sync_copy(y_s.at[pl.ds(o, _CROWS)], zbuf)
            pltpu.sync_copy(zbuf, yp0_hbm.at[pl.ds(o, _CROWS)])

    @pl.when(cid == 1)
    def _():
        for q in range(_CNQ):
            o = sid * _VSL + q * _CROWS
            pltpu.sync_copy(y_s.at[pl.ds(o, _CROWS)], zbuf)
            pltpu.sync_copy(zbuf, yp1_hbm.at[pl.ds(o, _CROWS)])


def _spmm_sc(x, erp, ecp, ewp):
    mesh = plsc.VectorSubcoreMesh(core_axis_name="c", subcore_axis_name="s")
    f = pl.kernel(
        _spmm_sc_body,
        out_type=(
            jax.ShapeDtypeStruct((_VP, B), jnp.float32),
            jax.ShapeDtypeStruct((_VP, B), jnp.float32),
        ),
        mesh=mesh,
        scratch_types=[
            pltpu.VMEM((_CCHUNK,), jnp.int32),
            pltpu.VMEM((_CCHUNK,), jnp.int32),
            pltpu.VMEM((_CCHUNK,), jnp.int32),
            pltpu.VMEM((_CCHUNK,), jnp.int32),
            pltpu.VMEM((_CCHUNK,), jnp.int32),
            pltpu.VMEM((_CEPT,), jnp.int32),
            pltpu.VMEM((_CEPT,), jnp.float32),
            pltpu.VMEM((_CCHUNK, B), jnp.float32),
            pltpu.VMEM((_CROWS, B), jnp.float32),
            pltpu.VMEM_SHARED((_VP, B), jnp.float32),
            pltpu.SemaphoreType.DMA,
        ],
        compiler_params=pltpu.CompilerParams(use_tc_tiling_on_sc=False),
    )
    return f(x, erp, ecp, ewp)


_CBLK = 2000


def _combine_body(yp0_ref, yp1_ref, xp_ref, xp2_ref, lm_ref, out_ref,
                  *, am, b1, b2):
    a = am / lm_ref[0, 0]
    out_ref[...] = (
        a * (yp0_ref[...] + yp1_ref[...])
        - b1 * xp_ref[...]
        - b2 * xp2_ref[...]
    )


def _combine(yp0, yp1, xp, xp2, lmax16, am, b1, b2):
    return pl.pallas_call(
        functools.partial(_combine_body, am=am, b1=b1, b2=b2),
        grid=(V // _CBLK,),
        in_specs=[
            pl.BlockSpec((_CBLK, B), lambda i: (i, 0)),
            pl.BlockSpec((_CBLK, B), lambda i: (i, 0)),
            pl.BlockSpec((_CBLK, B), lambda i: (i, 0)),
            pl.BlockSpec((_CBLK, B), lambda i: (i, 0)),
            pl.BlockSpec((1, 16), lambda i: (0, 0)),
        ],
        out_specs=pl.BlockSpec((_CBLK, B), lambda i: (i, 0)),
        out_shape=jax.ShapeDtypeStruct((V, B), jnp.float32),
    )(yp0, yp1, xp, xp2, lmax16)


def kernel(x_in, d, edge_index, edge_weight, W_cl1, b_cl1, W_fc1, b_fc1,
           W_fc2, b_fc2, W_fc3, b_fc3, W_nn1, b_nn1, W_nn2, b_nn2,
           W_sum2, b_sum2, W_im1, b_im1, W_im2, b_im2):
    x0 = x_in[:, :, 1].T
    x_nnT = x_in[:, :743, 0].T

    lmax16_arr = _power_lmax(edge_index[0], edge_index[1], edge_weight)

    lmax16 = lmax16_arr.reshape(1, 16)
    erp = jnp.pad(edge_index[0].reshape(32, E // 32), ((0, 0), (0, 120))).reshape(-1)
    ecp = jnp.pad(edge_index[1].reshape(32, E // 32), ((0, 0), (0, 120))).reshape(-1)
    ewp = jnp.pad(edge_weight.reshape(32, E // 32), ((0, 0), (0, 120))).reshape(-1)

    xs = [x0]
    xa, xb = x0, x0
    yp0, yp1 = _spmm_sc(x0, erp, ecp, ewp)
    for k in range(1, K):
        if k == 1:
            xn = _combine(yp0, yp1, xb, xb, lmax16, 2.0, 1.0, 0.0)
        else:
            xn = _combine(yp0, yp1, xb, xa, lmax16, 4.0, 2.0, 1.0)
        xs.append(xn)
        xa, xb = xb, xn
        if k < K - 1:
            yp0, yp1 = _spmm_sc(xn, erp, ecp, ewp)
    xc2 = jnp.stack(xs, 0).reshape(K, V * B)

    xpT = _cheby_pool(xc2, W_cl1, b_cl1)
    hT = _fc1(xpT, W_fc1, b_fc1)
    d2T = _fc2(hT, W_fc2, b_fc2)
    dec = _fc3(d2T, W_fc3, b_fc3)
    h, xn, xout, gae_pred, fc_pred = _heads(
        hT, x_nnT, W_nn1, b_nn1, W_nn2, b_nn2, W_sum2, b_sum2,
        W_im1, b_im1, W_im2, b_im2)
    return (dec, h, xout, xn, gae_pred, fc_pred)


# speedup vs baseline: 1.1783x; 1.1783x over previous
import functools

import jax
import jax.numpy as jnp
from jax import lax
from jax.experimental import pallas as pl
from jax.experimental.pallas import tpu as pltpu
from jax.experimental.pallas import tpu_sc as plsc

V = 10000
E = 160000
K = 5
CL1_F = 32
POOL = 8
B = 64
NPOOL = V // POOL
FC1_IN = NPOOL * CL1_F


_VBLK = 400
_PBLK = _VBLK // POOL


def _cheby_pool_body(xc_ref, w_ref, b_ref, out_ref):
    x = xc_ref[...]
    w = w_ref[...]
    pre = lax.dot_general(x, w, (((0,), (0,)), ((), ())))
    pre = pre.reshape(_PBLK, POOL, B, CL1_F)
    pooled = jnp.max(pre, axis=1)
    act = jnp.maximum(pooled + b_ref[...][None, :, :].reshape(1, 1, CL1_F), 0.0)
    out_ref[...] = jnp.transpose(act, (0, 2, 1))


def _cheby_pool(xc2, W_cl1, b_cl1):
    grid = V // _VBLK
    out = pl.pallas_call(
        _cheby_pool_body,
        grid=(grid,),
        in_specs=[
            pl.BlockSpec((K, _VBLK * B), lambda i: (0, i)),
            pl.BlockSpec((K, CL1_F), lambda i: (0, 0)),
            pl.BlockSpec((1, CL1_F), lambda i: (0, 0)),
        ],
        out_specs=pl.BlockSpec((_PBLK, CL1_F, B), lambda i: (i, 0, 0)),
        out_shape=jax.ShapeDtypeStruct((NPOOL, CL1_F, B), jnp.float32),
    )(xc2, W_cl1, b_cl1.reshape(1, CL1_F))
    return out.reshape(FC1_IN, B)


_IBLK = 2000


def _fc1_body(xp_ref, w_ref, b_ref, out_ref, acc_ref):
    @pl.when(pl.program_id(0) == 0)
    def _():
        acc_ref[...] = jnp.zeros_like(acc_ref)

    acc_ref[...] += lax.dot_general(
        w_ref[...], xp_ref[...], (((0,), (0,)), ((), ()))
    )

    @pl.when(pl.program_id(0) == pl.num_programs(0) - 1)
    def _():
        out_ref[...] = jnp.maximum(acc_ref[...] + b_ref[...], 0.0)


def _fc1(xpT, W_fc1, b_fc1):
    grid = FC1_IN // _IBLK
    return pl.pallas_call(
        _fc1_body,
        grid=(grid,),
        in_specs=[
            pl.BlockSpec((_IBLK, B), lambda i: (i, 0)),
            pl.BlockSpec((_IBLK, 512), lambda i: (i, 0)),
            pl.BlockSpec((512, 1), lambda i: (0, 0)),
        ],
        out_specs=pl.BlockSpec((512, B), lambda i: (0, 0)),
        out_shape=jax.ShapeDtypeStruct((512, B), jnp.float32),
        scratch_shapes=[pltpu.VMEM((512, B), jnp.float32)],
    )(xpT, W_fc1, b_fc1.reshape(512, 1))


def _fc2_body(h_ref, w_ref, b_ref, out_ref):
    out_ref[...] = jnp.maximum(
        lax.dot_general(w_ref[...], h_ref[...], (((0,), (0,)), ((), ())))
        + b_ref[...],
        0.0,
    )


def _fc2(hT, W_fc2, b_fc2):
    return pl.pallas_call(
        _fc2_body,
        out_shape=jax.ShapeDtypeStruct((1024, B), jnp.float32),
    )(hT, W_fc2, b_fc2.reshape(1024, 1))


_KBLK = 128


def _fc3_body(d_ref, w_ref, b_ref, out_ref, acc_ref):
    @pl.when(pl.program_id(0) == 0)
    def _():
        acc_ref[...] = jnp.zeros_like(acc_ref)

    acc_ref[...] += lax.dot_general(
        d_ref[...], w_ref[...], (((0,), (0,)), ((), ()))
    )

    @pl.when(pl.program_id(0) == pl.num_programs(0) - 1)
    def _():
        out_ref[...] = acc_ref[...] + b_ref[...]


def _fc3(d2T, W_fc3, b_fc3):
    grid = 1024 // _KBLK
    return pl.pallas_call(
        _fc3_body,
        grid=(grid,),
        in_specs=[
            pl.BlockSpec((_KBLK, B), lambda i: (i, 0)),
            pl.BlockSpec((_KBLK, V), lambda i: (i, 0)),
            pl.BlockSpec((1, V), lambda i: (0, 0)),
        ],
        out_specs=pl.BlockSpec((B, V), lambda i: (0, 0)),
        out_shape=jax.ShapeDtypeStruct((B, V), jnp.float32),
        scratch_shapes=[pltpu.VMEM((B, V), jnp.float32)],
    )(d2T, W_fc3, b_fc3.reshape(1, V))


def _log_softmax(x):
    m = jnp.max(x, axis=1, keepdims=True)
    s = x - m
    return s - jnp.log(jnp.sum(jnp.exp(s), axis=1, keepdims=True))


def _heads_body(h_ref, xnn_ref, wn1_ref, bn1_ref, wn2_ref, bn2_ref,
                ws2_ref, bs2_ref, wi1_ref, bi1_ref, wi2_ref, bi2_ref,
                h_out, xn_out, xout_out, gae_out, fc_out):
    hT = h_ref[...]
    a1 = jnp.maximum(
        lax.dot_general(wn1_ref[...], xnn_ref[...], (((0,), (0,)), ((), ())))
        + bn1_ref[...],
        0.0,
    )
    xnT = jnp.maximum(
        lax.dot_general(wn2_ref[...], a1, (((0,), (0,)), ((), ())))
        + bn2_ref[...],
        0.0,
    )
    h_out[...] = hT.T
    xn_out[...] = xnT.T
    xcatT = jnp.concatenate([hT, xnT], axis=0)
    lg = lax.dot_general(xcatT, ws2_ref[...], (((0,), (0,)), ((), ()))) + bs2_ref[...]
    xout_out[...] = _log_softmax(lg)
    g1 = lax.dot_general(hT, wi1_ref[...], (((0,), (0,)), ((), ()))) + bi1_ref[...]
    gae_out[...] = _log_softmax(jnp.dot(g1, wi2_ref[...]) + bi2_ref[...])
    f1 = lax.dot_general(xnT, wi1_ref[...], (((0,), (0,)), ((), ()))) + bi1_ref[...]
    fc_out[...] = _log_softmax(jnp.dot(f1, wi2_ref[...]) + bi2_ref[...])


def _heads(hT, x_nnT, W_nn1, b_nn1, W_nn2, b_nn2, W_sum2, b_sum2,
           W_im1, b_im1, W_im2, b_im2):
    return pl.pallas_call(
        _heads_body,
        out_shape=(
            jax.ShapeDtypeStruct((B, 512), jnp.float32),
            jax.ShapeDtypeStruct((B, 512), jnp.float32),
            jax.ShapeDtypeStruct((B, 27), jnp.float32),
            jax.ShapeDtypeStruct((B, 27), jnp.float32),
            jax.ShapeDtypeStruct((B, 27), jnp.float32),
        ),
    )(hT, x_nnT, W_nn1, b_nn1.reshape(1024, 1), W_nn2, b_nn2.reshape(512, 1),
      W_sum2, b_sum2.reshape(1, 27), W_im1, b_im1.reshape(1, 256),
      W_im2, b_im2.reshape(1, 27))


_NTILES = 16
_EPT = E // _NTILES
_VPAD = 10240
_SLICE = _VPAD // _NTILES
_POWER_ITERS = 50


def _inv_norm(n2):
    x = jnp.full((16,), n2, jnp.float32)

    def sb(i, s):
        return 0.5 * (s + x / s)
    s = lax.fori_loop(0, 34, sb, jnp.maximum(x, 1e-12))
    return 1.0 / (s + 1e-12)


def _power_body(er_hbm, ec_hbm, ew_hbm, lmax_hbm, rows_v, cols_v, w_v, gx_v,
                vals_v, sl_v, sl2_v, red_v, part_v, y0_s, y1_s, part_s, sem):
    cid = lax.axis_index("c")
    sid = lax.axis_index("s")
    base = sid * _EPT
    sbase = sid * _SLICE

    pltpu.sync_copy(er_hbm.at[pl.ds(base, _EPT)], rows_v)
    pltpu.sync_copy(ec_hbm.at[pl.ds(base, _EPT)], cols_v)
    pltpu.sync_copy(ew_hbm.at[pl.ds(base, _EPT)], w_v)

    def _fill(ref, n, val):
        def fb(i, _):
            ref[pl.ds(i * 16, 16)] = jnp.full((16,), val, jnp.float32)
            return 0
        lax.fori_loop(0, n // 16, fb, 0)

    _fill(sl_v, _SLICE, 1.0)
    pltpu.sync_copy(sl_v, y0_s.at[pl.ds(sbase, _SLICE)])
    _fill(sl_v, _SLICE, 0.0)
    plsc.subcore_barrier()

    def _spmm_step(ycur, ynext, inv):
        pltpu.sync_copy(sl_v, ynext.at[pl.ds(sbase, _SLICE)])
        pltpu.async_copy(ycur.at[cols_v], gx_v, sem).wait()

        def vb(i, _):
            o = i * 16
            vals_v[pl.ds(o, 16)] = gx_v[pl.ds(o, 16)] * w_v[pl.ds(o, 16)] * inv
            return 0
        lax.fori_loop(0, _EPT // 16, vb, 0)
        plsc.subcore_barrier()
        pltpu.sync_copy(vals_v, ynext.at[rows_v], add=True)
        plsc.subcore_barrier()

    def _sum_partials():
        pltpu.sync_copy(part_v, part_s.at[pl.ds(sid * 16, 16)])
        plsc.subcore_barrier()
        pltpu.sync_copy(part_s, red_v)

        def rb(i, acc):
            return acc + red_v[pl.ds(i * 16, 16)]
        tot = lax.fori_loop(0, _NTILES, rb, jnp.zeros((16,), jnp.float32))
        s = tot[0]
        for j in range(1, 16):
            s = s + tot[j]
        return s

    def _step(ycur, ynext, inv):
        _spmm_step(ycur, ynext, inv)
        pltpu.sync_copy(ynext.at[pl.ds(sbase, _SLICE)], sl2_v)

        def nb(i, acc):
            x = sl2_v[pl.ds(i * 16, 16)]
            return acc + x * x
        part_v[...] = lax.fori_loop(
            0, _SLICE // 16, nb, jnp.zeros((16,), jnp.float32))
        return _inv_norm(_sum_partials())

    def _pair(t, inv):
        inv = _step(y0_s, y1_s, inv)
        return _step(y1_s, y0_s, inv)

    inv = lax.fori_loop(0, _POWER_ITERS // 2, _pair,
                        jnp.ones((16,), jnp.float32))

    _spmm_step(y0_s, y1_s, inv)
    pltpu.sync_copy(y0_s.at[pl.ds(sbase, _SLICE)], sl2_v)
    pltpu.sync_copy(y1_s.at[pl.ds(sbase, _SLICE)], gx_v.at[pl.ds(0, _SLICE)])

    def db(i, acc):
        o = i * 16
        return acc + sl2_v[pl.ds(o, 16)] * inv * gx_v[pl.ds(o, 16)]
    part_v[...] = lax.fori_loop(
        0, _SLICE // 16, db, jnp.zeros((16,), jnp.float32))
    lmax = _sum_partials()

    @pl.when(jnp.logical_and(cid == 0, sid == 0))
    def _():
        part_v[...] = jnp.zeros((16,), jnp.float32) + lmax
        pltpu.sync_copy(part_v, lmax_hbm)


def _power_lmax(edge_rows, edge_cols, edge_weight):
    mesh = plsc.VectorSubcoreMesh(core_axis_name="c", subcore_axis_name="s")
    f = pl.kernel(
        _power_body,
        out_type=jax.ShapeDtypeStruct((16,), jnp.float32),
        mesh=mesh,
        scratch_types=[
            pltpu.VMEM((_EPT,), jnp.int32),
            pltpu.VMEM((_EPT,), jnp.int32),
            pltpu.VMEM((_EPT,), jnp.float32),
            pltpu.VMEM((_EPT,), jnp.float32),
            pltpu.VMEM((_EPT,), jnp.float32),
            pltpu.VMEM((_SLICE,), jnp.float32),
            pltpu.VMEM((_SLICE,), jnp.float32),
            pltpu.VMEM((_NTILES * 16,), jnp.float32),
            pltpu.VMEM((16,), jnp.float32),
            pltpu.VMEM_SHARED((_VPAD,), jnp.float32),
            pltpu.VMEM_SHARED((_VPAD,), jnp.float32),
            pltpu.VMEM_SHARED((_NTILES * 16,), jnp.float32),
            pltpu.SemaphoreType.DMA,
        ],
    )
    return f(edge_rows, edge_cols, edge_weight)


_VP = 10000
_VSL = _VP // 16
_CEPT = 5120
_CCHUNK = 1024
_CNCH = _CEPT // _CCHUNK
_CROWS = 125
_CNQ = _VSL // _CROWS


def _spmm_sc_body(x_hbm, er_hbm, ec_hbm, ew_hbm, yp0_hbm, yp1_hbm,
                  r0_v, r1_v, r2_v, r3_v, r4_v, cols_v, w_v, rowbuf, zbuf,
                  y_s, sem):
    cid = lax.axis_index("c")
    sid = lax.axis_index("s")
    ebase = (cid * 16 + sid) * _CEPT
    rows_refs = [r0_v, r1_v, r2_v, r3_v, r4_v]

    for c in range(_CNCH):
        pltpu.sync_copy(er_hbm.at[pl.ds(ebase + c * _CCHUNK, _CCHUNK)],
                        rows_refs[c])
    pltpu.sync_copy(ec_hbm.at[pl.ds(ebase, _CEPT)], cols_v)
    pltpu.sync_copy(ew_hbm.at[pl.ds(ebase, _CEPT)], w_v)

    def zb(r, _):
        for j in range(4):
            zbuf[r, pl.ds(j * 16, 16)] = jnp.zeros((16,), jnp.float32)
        return 0
    lax.fori_loop(0, _CROWS, zb, 0)
    for q in range(_CNQ):
        pltpu.sync_copy(zbuf, y_s.at[pl.ds(sid * _VSL + q * _CROWS, _CROWS)])
    plsc.subcore_barrier()

    for c in range(_CNCH):
        pltpu.async_copy(
            x_hbm.at[cols_v.at[pl.ds(c * _CCHUNK, _CCHUNK)]], rowbuf, sem
        ).wait()

        def gb(g, _):
            w16 = w_v[pl.ds(c * _CCHUNK + g * 16, 16)]
            r0 = g * 16
            for l in range(16):
                wl = jnp.zeros((16,), jnp.float32) + w16[l]
                for fb in range(4):
                    sl = pl.ds(fb * 16, 16)
                    rowbuf[r0 + l, sl] = rowbuf[r0 + l, sl] * wl
            return 0
        lax.fori_loop(0, _CCHUNK // 16, gb, 0)
        pltpu.sync_copy(rowbuf, y_s.at[rows_refs[c]], add=True)
    plsc.subcore_barrier()

    @pl.when(cid == 0)
    def _():
        for q in range(_CNQ):
            o = sid * _VSL + q * _CROWS
            pltpu.sync_copy(y_s.at[pl.ds(o, _CROWS)], zbuf)
            pltpu.sync_copy(zbuf, yp0_hbm.at[pl.ds(o, _CROWS)])

    @pl.when(cid == 1)
    def _():
        for q in range(_CNQ):
            o = sid * _VSL + q * _CROWS
            pltpu.sync_copy(y_s.at[pl.ds(o, _CROWS)], zbuf)
            pltpu.sync_copy(zbuf, yp1_hbm.at[pl.ds(o, _CROWS)])


def _spmm_sc(x, erp, ecp, ewp):
    mesh = plsc.VectorSubcoreMesh(core_axis_name="c", subcore_axis_name="s")
    f = pl.kernel(
        _spmm_sc_body,
        out_type=(
            jax.ShapeDtypeStruct((_VP, B), jnp.float32),
            jax.ShapeDtypeStruct((_VP, B), jnp.float32),
        ),
        mesh=mesh,
        scratch_types=[
            pltpu.VMEM((_CCHUNK,), jnp.int32),
            pltpu.VMEM((_CCHUNK,), jnp.int32),
            pltpu.VMEM((_CCHUNK,), jnp.int32),
            pltpu.VMEM((_CCHUNK,), jnp.int32),
            pltpu.VMEM((_CCHUNK,), jnp.int32),
            pltpu.VMEM((_CEPT,), jnp.int32),
            pltpu.VMEM((_CEPT,), jnp.float32),
            pltpu.VMEM((_CCHUNK, B), jnp.float32),
            pltpu.VMEM((_CROWS, B), jnp.float32),
            pltpu.VMEM_SHARED((_VP, B), jnp.float32),
            pltpu.SemaphoreType.DMA,
        ],
        compiler_params=pltpu.CompilerParams(use_tc_tiling_on_sc=False),
    )
    return f(x, erp, ecp, ewp)


_CBLK = 2000


def _combine_body(yp0_ref, yp1_ref, xp_ref, xp2_ref, lm_ref, out_ref,
                  *, am, b1, b2):
    a = am / lm_ref[0, 0]
    out_ref[...] = (
        a * (yp0_ref[...] + yp1_ref[...])
        - b1 * xp_ref[...]
        - b2 * xp2_ref[...]
    )


def _combine(yp0, yp1, xp, xp2, lmax16, am, b1, b2):
    return pl.pallas_call(
        functools.partial(_combine_body, am=am, b1=b1, b2=b2),
        grid=(V // _CBLK,),
        in_specs=[
            pl.BlockSpec((_CBLK, B), lambda i: (i, 0)),
            pl.BlockSpec((_CBLK, B), lambda i: (i, 0)),
            pl.BlockSpec((_CBLK, B), lambda i: (i, 0)),
            pl.BlockSpec((_CBLK, B), lambda i: (i, 0)),
            pl.BlockSpec((1, 16), lambda i: (0, 0)),
        ],
        out_specs=pl.BlockSpec((_CBLK, B), lambda i: (i, 0)),
        out_shape=jax.ShapeDtypeStruct((V, B), jnp.float32),
    )(yp0, yp1, xp, xp2, lmax16)


def kernel(x_in, d, edge_index, edge_weight, W_cl1, b_cl1, W_fc1, b_fc1,
           W_fc2, b_fc2, W_fc3, b_fc3, W_nn1, b_nn1, W_nn2, b_nn2,
           W_sum2, b_sum2, W_im1, b_im1, W_im2, b_im2):
    x0 = x_in[:, :, 1].T
    x_nnT = x_in[:, :743, 0].T

    lmax16_arr = _power_lmax(edge_index[0], edge_index[1], edge_weight)

    lmax16 = lmax16_arr.reshape(1, 16)
    erp = jnp.pad(edge_index[0].reshape(32, E // 32), ((0, 0), (0, 120))).reshape(-1)
    ecp = jnp.pad(edge_index[1].reshape(32, E // 32), ((0, 0), (0, 120))).reshape(-1)
    ewp = jnp.pad(edge_weight.reshape(32, E // 32), ((0, 0), (0, 120))).reshape(-1)

    xs = [x0]
    xa, xb = x0, x0
    yp0, yp1 = _spmm_sc(x0, erp, ecp, ewp)
    for k in range(1, K):
        if k == 1:
            xn = _combine(yp0, yp1, xb, xb, lmax16, 2.0, 1.0, 0.0)
        else:
            xn = _combine(yp0, yp1, xb, xa, lmax16, 4.0, 2.0, 1.0)
        xs.append(xn)
        xa, xb = xb, xn
        if k < K - 1:
            yp0, yp1 = _spmm_sc(xn, erp, ecp, ewp)
    xc2 = jnp.stack(xs, 0).reshape(K, V * B)

    xpT = _cheby_pool(xc2, W_cl1, b_cl1)
    hT = _fc1(xpT, W_fc1, b_fc1)
    d2T = _fc2(hT, W_fc2, b_fc2)
    dec = _fc3(d2T, W_fc3, b_fc3)
    h, xn, xout, gae_pred, fc_pred = _heads(
        hT, x_nnT, W_nn1, b_nn1, W_nn2, b_nn2, W_sum2, b_sum2,
        W_im1, b_im1, W_im2, b_im2)
    return (dec, h, xout, xn, gae_pred, fc_pred)


# speedup vs baseline: 1.2398x; 1.0522x over previous
import functools

import jax
import jax.numpy as jnp
from jax import lax
from jax.experimental import pallas as pl
from jax.experimental.pallas import tpu as pltpu
from jax.experimental.pallas import tpu_sc as plsc

V = 10000
E = 160000
K = 5
CL1_F = 32
POOL = 8
B = 64
NPOOL = V // POOL
FC1_IN = NPOOL * CL1_F


_VBLK = 400
_PBLK = _VBLK // POOL


def _cheby_pool_body(xc_ref, w_ref, b_ref, out_ref):
    x = xc_ref[...]
    w = w_ref[...]
    pre = lax.dot_general(x, w, (((0,), (0,)), ((), ())))
    pre = pre.reshape(_PBLK, POOL, B, CL1_F)
    pooled = jnp.max(pre, axis=1)
    act = jnp.maximum(pooled + b_ref[...][None, :, :].reshape(1, 1, CL1_F), 0.0)
    out_ref[...] = jnp.transpose(act, (0, 2, 1))


def _cheby_pool(xc2, W_cl1, b_cl1):
    grid = V // _VBLK
    out = pl.pallas_call(
        _cheby_pool_body,
        grid=(grid,),
        in_specs=[
            pl.BlockSpec((K, _VBLK * B), lambda i: (0, i)),
            pl.BlockSpec((K, CL1_F), lambda i: (0, 0)),
            pl.BlockSpec((1, CL1_F), lambda i: (0, 0)),
        ],
        out_specs=pl.BlockSpec((_PBLK, CL1_F, B), lambda i: (i, 0, 0)),
        out_shape=jax.ShapeDtypeStruct((NPOOL, CL1_F, B), jnp.float32),
    )(xc2, W_cl1, b_cl1.reshape(1, CL1_F))
    return out.reshape(FC1_IN, B)


_IBLK = 2000


def _fc1_body(xp_ref, w_ref, b_ref, out_ref, acc_ref):
    @pl.when(pl.program_id(0) == 0)
    def _():
        acc_ref[...] = jnp.zeros_like(acc_ref)

    acc_ref[...] += lax.dot_general(
        w_ref[...], xp_ref[...], (((0,), (0,)), ((), ()))
    )

    @pl.when(pl.program_id(0) == pl.num_programs(0) - 1)
    def _():
        out_ref[...] = jnp.maximum(acc_ref[...] + b_ref[...], 0.0)


def _fc1(xpT, W_fc1, b_fc1):
    grid = FC1_IN // _IBLK
    return pl.pallas_call(
        _fc1_body,
        grid=(grid,),
        in_specs=[
            pl.BlockSpec((_IBLK, B), lambda i: (i, 0)),
            pl.BlockSpec((_IBLK, 512), lambda i: (i, 0)),
            pl.BlockSpec((512, 1), lambda i: (0, 0)),
        ],
        out_specs=pl.BlockSpec((512, B), lambda i: (0, 0)),
        out_shape=jax.ShapeDtypeStruct((512, B), jnp.float32),
        scratch_shapes=[pltpu.VMEM((512, B), jnp.float32)],
    )(xpT, W_fc1, b_fc1.reshape(512, 1))


def _fc2_body(h_ref, w_ref, b_ref, out_ref):
    out_ref[...] = jnp.maximum(
        lax.dot_general(w_ref[...], h_ref[...], (((0,), (0,)), ((), ())))
        + b_ref[...],
        0.0,
    )


def _fc2(hT, W_fc2, b_fc2):
    return pl.pallas_call(
        _fc2_body,
        out_shape=jax.ShapeDtypeStruct((1024, B), jnp.float32),
    )(hT, W_fc2, b_fc2.reshape(1024, 1))


_KBLK = 128


def _fc3_body(d_ref, w_ref, b_ref, out_ref, acc_ref):
    @pl.when(pl.program_id(0) == 0)
    def _():
        acc_ref[...] = jnp.zeros_like(acc_ref)

    acc_ref[...] += lax.dot_general(
        d_ref[...], w_ref[...], (((0,), (0,)), ((), ()))
    )

    @pl.when(pl.program_id(0) == pl.num_programs(0) - 1)
    def _():
        out_ref[...] = acc_ref[...] + b_ref[...]


def _fc3(d2T, W_fc3, b_fc3):
    grid = 1024 // _KBLK
    return pl.pallas_call(
        _fc3_body,
        grid=(grid,),
        in_specs=[
            pl.BlockSpec((_KBLK, B), lambda i: (i, 0)),
            pl.BlockSpec((_KBLK, V), lambda i: (i, 0)),
            pl.BlockSpec((1, V), lambda i: (0, 0)),
        ],
        out_specs=pl.BlockSpec((B, V), lambda i: (0, 0)),
        out_shape=jax.ShapeDtypeStruct((B, V), jnp.float32),
        scratch_shapes=[pltpu.VMEM((B, V), jnp.float32)],
    )(d2T, W_fc3, b_fc3.reshape(1, V))


def _log_softmax(x):
    m = jnp.max(x, axis=1, keepdims=True)
    s = x - m
    return s - jnp.log(jnp.sum(jnp.exp(s), axis=1, keepdims=True))


def _heads_body(h_ref, xnn_ref, wn1_ref, bn1_ref, wn2_ref, bn2_ref,
                ws2_ref, bs2_ref, wi1_ref, bi1_ref, wi2_ref, bi2_ref,
                h_out, xn_out, xout_out, gae_out, fc_out):
    hT = h_ref[...]
    a1 = jnp.maximum(
        lax.dot_general(wn1_ref[...], xnn_ref[...], (((0,), (0,)), ((), ())))
        + bn1_ref[...],
        0.0,
    )
    xnT = jnp.maximum(
        lax.dot_general(wn2_ref[...], a1, (((0,), (0,)), ((), ())))
        + bn2_ref[...],
        0.0,
    )
    h_out[...] = hT.T
    xn_out[...] = xnT.T
    xcatT = jnp.concatenate([hT, xnT], axis=0)
    lg = lax.dot_general(xcatT, ws2_ref[...], (((0,), (0,)), ((), ()))) + bs2_ref[...]
    xout_out[...] = _log_softmax(lg)
    g1 = lax.dot_general(hT, wi1_ref[...], (((0,), (0,)), ((), ()))) + bi1_ref[...]
    gae_out[...] = _log_softmax(jnp.dot(g1, wi2_ref[...]) + bi2_ref[...])
    f1 = lax.dot_general(xnT, wi1_ref[...], (((0,), (0,)), ((), ()))) + bi1_ref[...]
    fc_out[...] = _log_softmax(jnp.dot(f1, wi2_ref[...]) + bi2_ref[...])


def _heads(hT, x_nnT, W_nn1, b_nn1, W_nn2, b_nn2, W_sum2, b_sum2,
           W_im1, b_im1, W_im2, b_im2):
    return pl.pallas_call(
        _heads_body,
        out_shape=(
            jax.ShapeDtypeStruct((B, 512), jnp.float32),
            jax.ShapeDtypeStruct((B, 512), jnp.float32),
            jax.ShapeDtypeStruct((B, 27), jnp.float32),
            jax.ShapeDtypeStruct((B, 27), jnp.float32),
            jax.ShapeDtypeStruct((B, 27), jnp.float32),
        ),
    )(hT, x_nnT, W_nn1, b_nn1.reshape(1024, 1), W_nn2, b_nn2.reshape(512, 1),
      W_sum2, b_sum2.reshape(1, 27), W_im1, b_im1.reshape(1, 256),
      W_im2, b_im2.reshape(1, 27))


_NTILES = 16
_EPT = E // _NTILES
_VPAD = 10240
_SLICE = _VPAD // _NTILES
_POWER_ITERS = 50


def _inv_norm(n2):
    x = jnp.full((16,), n2, jnp.float32)

    def sb(i, s):
        return 0.5 * (s + x / s)
    s = lax.fori_loop(0, 34, sb, jnp.maximum(x, 1e-12))
    return 1.0 / (s + 1e-12)


def _power_body(er_hbm, ec_hbm, ew_hbm, lmax_hbm, rows_v, cols_v, w_v, gx_v,
                vals_v, sl_v, sl2_v, red_v, part_v, y0_s, y1_s, part_s, sem):
    cid = lax.axis_index("c")
    sid = lax.axis_index("s")
    base = sid * _EPT
    sbase = sid * _SLICE

    pltpu.sync_copy(er_hbm.at[pl.ds(base, _EPT)], rows_v)
    pltpu.sync_copy(ec_hbm.at[pl.ds(base, _EPT)], cols_v)
    pltpu.sync_copy(ew_hbm.at[pl.ds(base, _EPT)], w_v)

    def _fill(ref, n, val):
        def fb(i, _):
            ref[pl.ds(i * 16, 16)] = jnp.full((16,), val, jnp.float32)
            return 0
        lax.fori_loop(0, n // 16, fb, 0)

    _fill(sl_v, _SLICE, 1.0)
    pltpu.sync_copy(sl_v, y0_s.at[pl.ds(sbase, _SLICE)])
    _fill(sl_v, _SLICE, 0.0)
    plsc.subcore_barrier()

    def _spmm_step(ycur, ynext, inv):
        pltpu.sync_copy(sl_v, ynext.at[pl.ds(sbase, _SLICE)])
        pltpu.async_copy(ycur.at[cols_v], gx_v, sem).wait()

        def vb(i, _):
            o = i * 16
            vals_v[pl.ds(o, 16)] = gx_v[pl.ds(o, 16)] * w_v[pl.ds(o, 16)] * inv
            return 0
        lax.fori_loop(0, _EPT // 16, vb, 0)
        plsc.subcore_barrier()
        pltpu.sync_copy(vals_v, ynext.at[rows_v], add=True)
        plsc.subcore_barrier()

    def _sum_partials():
        pltpu.sync_copy(part_v, part_s.at[pl.ds(sid * 16, 16)])
        plsc.subcore_barrier()
        pltpu.sync_copy(part_s, red_v)

        def rb(i, acc):
            return acc + red_v[pl.ds(i * 16, 16)]
        tot = lax.fori_loop(0, _NTILES, rb, jnp.zeros((16,), jnp.float32))
        s = tot[0]
        for j in range(1, 16):
            s = s + tot[j]
        return s

    def _step(ycur, ynext, inv):
        _spmm_step(ycur, ynext, inv)
        pltpu.sync_copy(ynext.at[pl.ds(sbase, _SLICE)], sl2_v)

        def nb(i, acc):
            x = sl2_v[pl.ds(i * 16, 16)]
            return acc + x * x
        part_v[...] = lax.fori_loop(
            0, _SLICE // 16, nb, jnp.zeros((16,), jnp.float32))
        return _inv_norm(_sum_partials())

    def _pair(t, inv):
        inv = _step(y0_s, y1_s, inv)
        return _step(y1_s, y0_s, inv)

    inv = lax.fori_loop(0, _POWER_ITERS // 2, _pair,
                        jnp.ones((16,), jnp.float32))

    _spmm_step(y0_s, y1_s, inv)
    pltpu.sync_copy(y0_s.at[pl.ds(sbase, _SLICE)], sl2_v)
    pltpu.sync_copy(y1_s.at[pl.ds(sbase, _SLICE)], gx_v.at[pl.ds(0, _SLICE)])

    def db(i, acc):
        o = i * 16
        return acc + sl2_v[pl.ds(o, 16)] * inv * gx_v[pl.ds(o, 16)]
    part_v[...] = lax.fori_loop(
        0, _SLICE // 16, db, jnp.zeros((16,), jnp.float32))
    lmax = _sum_partials()

    @pl.when(jnp.logical_and(cid == 0, sid == 0))
    def _():
        part_v[...] = jnp.zeros((16,), jnp.float32) + lmax
        pltpu.sync_copy(part_v, lmax_hbm)


def _power_lmax(edge_rows, edge_cols, edge_weight):
    mesh = plsc.VectorSubcoreMesh(core_axis_name="c", subcore_axis_name="s")
    f = pl.kernel(
        _power_body,
        out_type=jax.ShapeDtypeStruct((16,), jnp.float32),
        mesh=mesh,
        scratch_types=[
            pltpu.VMEM((_EPT,), jnp.int32),
            pltpu.VMEM((_EPT,), jnp.int32),
            pltpu.VMEM((_EPT,), jnp.float32),
            pltpu.VMEM((_EPT,), jnp.float32),
            pltpu.VMEM((_EPT,), jnp.float32),
            pltpu.VMEM((_SLICE,), jnp.float32),
            pltpu.VMEM((_SLICE,), jnp.float32),
            pltpu.VMEM((_NTILES * 16,), jnp.float32),
            pltpu.VMEM((16,), jnp.float32),
            pltpu.VMEM_SHARED((_VPAD,), jnp.float32),
            pltpu.VMEM_SHARED((_VPAD,), jnp.float32),
            pltpu.VMEM_SHARED((_NTILES * 16,), jnp.float32),
            pltpu.SemaphoreType.DMA,
        ],
    )
    return f(edge_rows, edge_cols, edge_weight)


_VP = 10000
_VSL = _VP // 16
_CEPT = 5120
_CCHUNK = 512
_CNCH = _CEPT // _CCHUNK
_CROWS = 125
_CNQ = _VSL // _CROWS


def _spmm_sc_body(x_hbm, er_hbm, ec_hbm, ew_hbm, yp0_hbm, yp1_hbm,
                  r0_v, r1_v, r2_v, r3_v, r4_v, r5_v, r6_v, r7_v, r8_v, r9_v,
                  cols_v, w_v, bufa, bufb, zbuf, y_s, semg, sems):
    cid = lax.axis_index("c")
    sid = lax.axis_index("s")
    ebase = (cid * 16 + sid) * _CEPT
    rows_refs = [r0_v, r1_v, r2_v, r3_v, r4_v, r5_v, r6_v, r7_v, r8_v, r9_v]
    bufs = [bufa, bufb]

    for c in range(_CNCH):
        pltpu.sync_copy(er_hbm.at[pl.ds(ebase + c * _CCHUNK, _CCHUNK)],
                        rows_refs[c])
    pltpu.sync_copy(ec_hbm.at[pl.ds(ebase, _CEPT)], cols_v)
    pltpu.sync_copy(ew_hbm.at[pl.ds(ebase, _CEPT)], w_v)

    def zb(r, _):
        for j in range(4):
            zbuf[r, pl.ds(j * 16, 16)] = jnp.zeros((16,), jnp.float32)
        return 0
    lax.fori_loop(0, _CROWS, zb, 0)
    for q in range(_CNQ):
        pltpu.sync_copy(zbuf, y_s.at[pl.ds(sid * _VSL + q * _CROWS, _CROWS)])
    plsc.subcore_barrier()

    def _gather(c):
        return pltpu.async_copy(
            x_hbm.at[cols_v.at[pl.ds(c * _CCHUNK, _CCHUNK)]],
            bufs[c % 2], semg)

    gd = {0: _gather(0)}
    sd = {}
    for c in range(_CNCH):
        buf = bufs[c % 2]
        if c < _CNCH - 1:
            if c >= 1:
                sd[c - 1].wait()
            gd[c + 1] = _gather(c + 1)
        gd[c].wait()

        def gb(g, _):
            w16 = w_v[pl.ds(c * _CCHUNK + g * 16, 16)]
            r0 = g * 16
            for l in range(16):
                wl = jnp.zeros((16,), jnp.float32) + w16[l]
                for fb in range(4):
                    sl = pl.ds(fb * 16, 16)
                    buf[r0 + l, sl] = buf[r0 + l, sl] * wl
            return 0
        lax.fori_loop(0, _CCHUNK // 16, gb, 0)
        sd[c] = pltpu.async_copy(buf, y_s.at[rows_refs[c]], sems, add=True)
    sd[_CNCH - 2].wait()
    sd[_CNCH - 1].wait()
    plsc.subcore_barrier()

    @pl.when(cid == 0)
    def _():
        for q in range(_CNQ):
            o = sid * _VSL + q * _CROWS
            pltpu.sync_copy(y_s.at[pl.ds(o, _CROWS)], zbuf)
            pltpu.sync_copy(zbuf, yp0_hbm.at[pl.ds(o, _CROWS)])

    @pl.when(cid == 1)
    def _():
        for q in range(_CNQ):
            o = sid * _VSL + q * _CROWS
            pltpu.sync_copy(y_s.at[pl.ds(o, _CROWS)], zbuf)
            pltpu.sync_copy(zbuf, yp1_hbm.at[pl.ds(o, _CROWS)])


def _spmm_sc(x, erp, ecp, ewp):
    mesh = plsc.VectorSubcoreMesh(core_axis_name="c", subcore_axis_name="s")
    f = pl.kernel(
        _spmm_sc_body,
        out_type=(
            jax.ShapeDtypeStruct((_VP, B), jnp.float32),
            jax.ShapeDtypeStruct((_VP, B), jnp.float32),
        ),
        mesh=mesh,
        scratch_types=(
            [pltpu.VMEM((_CCHUNK,), jnp.int32) for _ in range(_CNCH)]
            + [
                pltpu.VMEM((_CEPT,), jnp.int32),
                pltpu.VMEM((_CEPT,), jnp.float32),
                pltpu.VMEM((_CCHUNK, B), jnp.float32),
                pltpu.VMEM((_CCHUNK, B), jnp.float32),
                pltpu.VMEM((_CROWS, B), jnp.float32),
                pltpu.VMEM_SHARED((_VP, B), jnp.float32),
                pltpu.SemaphoreType.DMA,
                pltpu.SemaphoreType.DMA,
            ]
        ),
        compiler_params=pltpu.CompilerParams(use_tc_tiling_on_sc=False),
    )
    return f(x, erp, ecp, ewp)


_CBLK = 2000


def _combine_body(yp0_ref, yp1_ref, xp_ref, xp2_ref, lm_ref, out_ref,
                  *, am, b1, b2):
    a = am / lm_ref[0, 0]
    out_ref[...] = (
        a * (yp0_ref[...] + yp1_ref[...])
        - b1 * xp_ref[...]
        - b2 * xp2_ref[...]
    )


def _combine(yp0, yp1, xp, xp2, lmax16, am, b1, b2):
    return pl.pallas_call(
        functools.partial(_combine_body, am=am, b1=b1, b2=b2),
        grid=(V // _CBLK,),
        in_specs=[
            pl.BlockSpec((_CBLK, B), lambda i: (i, 0)),
            pl.BlockSpec((_CBLK, B), lambda i: (i, 0)),
            pl.BlockSpec((_CBLK, B), lambda i: (i, 0)),
            pl.BlockSpec((_CBLK, B), lambda i: (i, 0)),
            pl.BlockSpec((1, 16), lambda i: (0, 0)),
        ],
        out_specs=pl.BlockSpec((_CBLK, B), lambda i: (i, 0)),
        out_shape=jax.ShapeDtypeStruct((V, B), jnp.float32),
    )(yp0, yp1, xp, xp2, lmax16)


def kernel(x_in, d, edge_index, edge_weight, W_cl1, b_cl1, W_fc1, b_fc1,
           W_fc2, b_fc2, W_fc3, b_fc3, W_nn1, b_nn1, W_nn2, b_nn2,
           W_sum2, b_sum2, W_im1, b_im1, W_im2, b_im2):
    x0 = x_in[:, :, 1].T
    x_nnT = x_in[:, :743, 0].T

    lmax16_arr = _power_lmax(edge_index[0], edge_index[1], edge_weight)

    lmax16 = lmax16_arr.reshape(1, 16)
    erp = jnp.pad(edge_index[0].reshape(32, E // 32), ((0, 0), (0, 120))).reshape(-1)
    ecp = jnp.pad(edge_index[1].reshape(32, E // 32), ((0, 0), (0, 120))).reshape(-1)
    ewp = jnp.pad(edge_weight.reshape(32, E // 32), ((0, 0), (0, 120))).reshape(-1)

    xs = [x0]
    xa, xb = x0, x0
    yp0, yp1 = _spmm_sc(x0, erp, ecp, ewp)
    for k in range(1, K):
        if k == 1:
            xn = _combine(yp0, yp1, xb, xb, lmax16, 2.0, 1.0, 0.0)
        else:
            xn = _combine(yp0, yp1, xb, xa, lmax16, 4.0, 2.0, 1.0)
        xs.append(xn)
        xa, xb = xb, xn
        if k < K - 1:
            yp0, yp1 = _spmm_sc(xn, erp, ecp, ewp)
    xc2 = jnp.stack(xs, 0).reshape(K, V * B)

    xpT = _cheby_pool(xc2, W_cl1, b_cl1)
    hT = _fc1(xpT, W_fc1, b_fc1)
    d2T = _fc2(hT, W_fc2, b_fc2)
    dec = _fc3(d2T, W_fc3, b_fc3)
    h, xn, xout, gae_pred, fc_pred = _heads(
        hT, x_nnT, W_nn1, b_nn1, W_nn2, b_nn2, W_sum2, b_sum2,
        W_im1, b_im1, W_im2, b_im2)
    return (dec, h, xout, xn, gae_pred, fc_pred)
